# baseline (speedup 1.0000x reference)
import functools

import jax
import jax.numpy as jnp
from jax import lax
from jax.experimental import pallas as pl
from jax.experimental.pallas import tpu as pltpu
from jax.experimental.pallas import tpu_sc as plsc

N = 8192
D_IN = 768
E = 8
HID = 256
D_OUT = 256
TOPK = 2

BR = 256
R_PAD = 18432
NBLK = R_PAD // BR
BT = 1024

NW = 32
GCH = 64
GNC = R_PAD // NW // GCH


def _gate_body(x_ref, wg1_ref, wg2_ref,
               a1_ref, a2_ref, w1_ref, w2_ref, r1_ref, r2_ref, cnt_ref,
               acc_ref):
    i = pl.program_id(0)

    @pl.when(i == 0)
    def _():
        acc_ref[...] = jnp.zeros_like(acc_ref)

    t = jnp.tanh(jnp.dot(x_ref[...], wg1_ref[...],
                         preferred_element_type=jnp.float32))
    logits = jnp.dot(t, wg2_ref[...], preferred_element_type=jnp.float32)
    m = jnp.max(logits, axis=-1, keepdims=True)
    ex = jnp.exp(logits - m)
    gw = ex / jnp.sum(ex, axis=-1, keepdims=True)
    a1 = jnp.argmax(gw, axis=-1)
    m1 = jnp.max(gw, axis=-1)
    lane = lax.broadcasted_iota(jnp.int32, gw.shape, 1)
    gw2 = jnp.where(lane == a1[:, None], -1.0, gw)
    a2 = jnp.argmax(gw2, axis=-1)
    m2 = jnp.max(gw2, axis=-1)
    s = m1 + m2 + 1e-12
    a1_ref[...] = a1.astype(jnp.int32)
    a2_ref[...] = a2.astype(jnp.int32)
    w1_ref[...] = m1 / s
    w2_ref[...] = m2 / s

    oh1 = (lane == a1[:, None]).astype(jnp.float32)
    oh2 = (lane == a2[:, None]).astype(jnp.float32)
    S = oh1 + oh2
    row = lax.broadcasted_iota(jnp.int32, (BT, BT), 0)
    col = lax.broadcasted_iota(jnp.int32, (BT, BT), 1)
    tri = (row > col).astype(jnp.float32)
    cex = jnp.dot(tri, S, preferred_element_type=jnp.float32)
    base = cex + acc_ref[...]
    r1_ref[...] = jnp.sum(base * oh1, axis=1).astype(jnp.int32)
    r2_ref[...] = jnp.sum(base * oh2, axis=1).astype(jnp.int32)
    new_acc = acc_ref[...] + jnp.sum(S, axis=0, keepdims=True)
    acc_ref[...] = new_acc
    cnt_ref[...] = new_acc.astype(jnp.int32)


def _gating(x, Wg1, Wg2):
    vec = lambda: pl.BlockSpec((BT,), lambda i: (i,))
    return pl.pallas_call(
        _gate_body,
        grid=(N // BT,),
        in_specs=[
            pl.BlockSpec((BT, D_IN), lambda i: (i, 0)),
            pl.BlockSpec((D_IN, 2 * E), lambda i: (0, 0)),
            pl.BlockSpec((2 * E, E), lambda i: (0, 0)),
        ],
        out_specs=[
            vec(), vec(), vec(), vec(), vec(), vec(),
            pl.BlockSpec((1, E), lambda i: (0, 0)),
        ],
        out_shape=[
            jax.ShapeDtypeStruct((N,), jnp.int32),
            jax.ShapeDtypeStruct((N,), jnp.int32),
            jax.ShapeDtypeStruct((N,), jnp.float32),
            jax.ShapeDtypeStruct((N,), jnp.float32),
            jax.ShapeDtypeStruct((N,), jnp.int32),
            jax.ShapeDtypeStruct((N,), jnp.int32),
            jax.ShapeDtypeStruct((1, E), jnp.int32),
        ],
        scratch_shapes=[pltpu.VMEM((1, E), jnp.float32)],
    )(x, Wg1, Wg2)


def _dest_body(a1_ref, a2_ref, r1_ref, r2_ref, cnt_ref,
               p1_ref, p2_ref, eid_ref):
    counts = cnt_ref[...]
    bpe = (counts + (BR - 1)) >> 8
    er = lax.broadcasted_iota(jnp.int32, (E, E), 0)
    ec = lax.broadcasted_iota(jnp.int32, (E, E), 1)
    etri = (er < ec).astype(jnp.float32)
    pad_start_b = jnp.dot(bpe.astype(jnp.float32), etri,
                          preferred_element_type=jnp.float32)
    pad_start = pad_start_b.astype(jnp.int32) * BR

    lane1 = lax.broadcasted_iota(jnp.int32, (N, E), 1)
    oh1 = (lane1 == a1_ref[...][:, None]).astype(jnp.int32)
    oh2 = (lane1 == a2_ref[...][:, None]).astype(jnp.int32)
    p1_ref[...] = r1_ref[...] + jnp.sum(oh1 * pad_start, axis=1)
    p2_ref[...] = r2_ref[...] + jnp.sum(oh2 * pad_start, axis=1)

    pad_end = pad_start + bpe * BR
    blk = lax.broadcasted_iota(jnp.int32, (NBLK, E), 0) * BR
    eid = jnp.sum((blk >= pad_end).astype(jnp.int32), axis=1)
    eid_ref[...] = jnp.minimum(eid, E - 1)


def _slot_dest(a1, a2, r1, r2, counts):
    def full(s):
        return pl.BlockSpec(s, lambda: tuple(0 for _ in s))
    return pl.pallas_call(
        _dest_body,
        in_specs=[full((N,)), full((N,)), full((N,)), full((N,)),
                  full((1, E))],
        out_specs=[full((N,)), full((N,)), full((NBLK,))],
        out_shape=[
            jax.ShapeDtypeStruct((N,), jnp.int32),
            jax.ShapeDtypeStruct((N,), jnp.int32),
            jax.ShapeDtypeStruct((NBLK,), jnp.int32),
        ],
    )(a1, a2, r1, r2, counts)


@functools.lru_cache(maxsize=None)
def _sc_gather_x_kernel():
    @functools.partial(
        pl.kernel,
        out_type=jax.ShapeDtypeStruct((R_PAD, D_IN), jnp.float32),
        mesh=plsc.VectorSubcoreMesh(core_axis_name="c", subcore_axis_name="s"),
        scratch_types=[
            pltpu.VMEM((GNC, GCH), jnp.int32),
            pltpu.VMEM((GCH, D_IN), jnp.float32),
            pltpu.VMEM((GCH, D_IN), jnp.float32),
            pltpu.SemaphoreType.DMA,
            pltpu.SemaphoreType.DMA,
            pltpu.SemaphoreType.DMA,
            pltpu.SemaphoreType.DMA,
        ],
    )
    def _sc_gather_x(tok_hbm, x_hbm, out_hbm, idx_v, buf0, buf1,
                     sg0, sg1, sw0, sw1):
        wid = lax.axis_index("s") * 2 + lax.axis_index("c")
        per_w = R_PAD // NW
        base = wid * per_w
        pltpu.sync_copy(tok_hbm.at[wid], idx_v)
        bufs = (buf0, buf1)
        gsem = (sg0, sg1)
        wsem = (sw0, sw1)
        g = [None] * GNC
        w = [None] * GNC

        def start_gather(ci):
            g[ci] = pltpu.async_copy(x_hbm.at[idx_v.at[ci]],
                                     bufs[ci % 2], gsem[ci % 2])

        start_gather(0)
        for ci in range(GNC):
            if ci + 1 < GNC:
                if ci + 1 >= 2:
                    w[ci - 1].wait()
                start_gather(ci + 1)
            g[ci].wait()
            w[ci] = pltpu.async_copy(
                bufs[ci % 2], out_hbm.at[pl.ds(base + ci * GCH, GCH)],
                wsem[ci % 2])
        w[GNC - 2].wait()
        w[GNC - 1].wait()

    return _sc_gather_x


def _ln(h):
    mu = jnp.mean(h, axis=-1, keepdims=True)
    var = jnp.mean((h - mu) ** 2, axis=-1, keepdims=True)
    return (h - mu) * lax.rsqrt(var + 1e-5)


def _gelu(h):
    return 0.5 * h * (1.0 + lax.erf(h * (2.0 ** -0.5)))


def _mlp_body(eid_ref, xg_ref, w_ref, W1_ref, b1_ref, g1_ref, be1_ref,
              W2_ref, b2_ref, g2_ref, be2_ref, W3_ref, b3_ref, o_ref):
    h = jnp.dot(xg_ref[...], W1_ref[0], preferred_element_type=jnp.float32)
    h = h + b1_ref[0]
    h = _ln(h) * g1_ref[0] + be1_ref[0]
    h = _gelu(h)
    h = jnp.dot(h, W2_ref[0], preferred_element_type=jnp.float32) + b2_ref[0]
    h = _ln(h) * g2_ref[0] + be2_ref[0]
    h = _gelu(h)
    h = jnp.dot(h, W3_ref[0], preferred_element_type=jnp.float32) + b3_ref[0]
    o_ref[...] = h * w_ref[...]


def _grouped_mlp(block_eid, xg, slot_w, W1, b1, g1, be1, W2, b2, g2, be2, W3, b3):
    def we(block_shape):
        n = len(block_shape)
        return pl.BlockSpec((1,) + block_shape,
                            lambda i, eid, _n=n: (eid[i],) + (0,) * _n)

    grid_spec = pltpu.PrefetchScalarGridSpec(
        num_scalar_prefetch=1,
        grid=(NBLK,),
        in_specs=[
            pl.BlockSpec((BR, D_IN), lambda i, eid: (i, 0)),
            pl.BlockSpec((BR, 1), lambda i, eid: (i, 0)),
            we((D_IN, HID)), we((1, HID)), we((1, HID)), we((1, HID)),
            we((HID, HID)), we((1, HID)), we((1, HID)), we((1, HID)),
            we((HID, D_OUT)), we((1, D_OUT)),
        ],
        out_specs=pl.BlockSpec((BR, D_OUT), lambda i, eid: (i, 0)),
    )
    r3 = lambda a: a.reshape(E, 1, a.shape[-1])
    return pl.pallas_call(
        _mlp_body,
        grid_spec=grid_spec,
        out_shape=jax.ShapeDtypeStruct((R_PAD, D_OUT), jnp.float32),
    )(block_eid, xg, slot_w.reshape(R_PAD, 1),
      W1, r3(b1), r3(g1), r3(be1), W2, r3(b2), r3(g2), r3(be2), W3, r3(b3))


CCH = 64


@functools.lru_cache(maxsize=None)
def _sc_combine_kernel():
    @functools.partial(
        pl.kernel,
        out_type=jax.ShapeDtypeStruct((N, D_OUT), jnp.float32),
        mesh=plsc.VectorSubcoreMesh(core_axis_name="c", subcore_axis_name="s"),
        scratch_types=[
            pltpu.VMEM((CCH,), jnp.int32),
            pltpu.VMEM((CCH,), jnp.int32),
            pltpu.VMEM((CCH, D_OUT), jnp.float32),
            pltpu.VMEM((CCH, D_OUT), jnp.float32),
            pltpu.SemaphoreType.DMA,
            pltpu.SemaphoreType.DMA,
        ],
    )
    def _sc_combine(p1_hbm, p2_hbm, y_hbm, out_hbm,
                    i1_v, i2_v, ra_v, rb_v, sem_a, sem_b):
        wid = lax.axis_index("s") * 2 + lax.axis_index("c")
        per_w = N // NW
        base = wid * per_w
        for ci in range(per_w // CCH):
            off = base + ci * CCH
            pltpu.sync_copy(p1_hbm.at[pl.ds(off, CCH)], i1_v)
            pltpu.sync_copy(p2_hbm.at[pl.ds(off, CCH)], i2_v)
            cpa = pltpu.async_copy(y_hbm.at[i1_v], ra_v, sem_a)
            cpb = pltpu.async_copy(y_hbm.at[i2_v], rb_v, sem_b)
            cpa.wait()
            cpb.wait()

            def row_body(r, _):
                for c in range(D_OUT // 16):
                    sl = pl.ds(c * 16, 16)
                    ra_v[r, sl] = ra_v[r, sl] + rb_v[r, sl]
                return 0

            lax.fori_loop(0, CCH, row_body, 0)
            pltpu.sync_copy(ra_v, out_hbm.at[pl.ds(off, CCH)])

    return _sc_combine


def kernel(x, Wg1, Wg2, W1, b1, g1, be1, W2, b2, g2, be2, W3, b3):
    a1, a2, w1, w2, r1, r2, counts = _gating(x, Wg1, Wg2)
    p1, p2, block_eid = _slot_dest(a1, a2, r1, r2, counts)
    tok = jnp.arange(N, dtype=jnp.int32)
    slot_tok = (jnp.zeros((R_PAD,), jnp.int32).at[p1].set(tok)
                .at[p2].set(tok)).reshape(NW, GNC, GCH)
    slot_w = jnp.zeros((R_PAD,), jnp.float32).at[p1].set(w1).at[p2].set(w2)
    xg = _sc_gather_x_kernel()(slot_tok, x)
    y = _grouped_mlp(block_eid, xg, slot_w,
                     W1, b1, g1, be1, W2, b2, g2, be2, W3, b3)
    return _sc_combine_kernel()(p1, p2, y)


# speedup vs baseline: 2.8192x; 2.8192x over previous
import jax
import jax.numpy as jnp
from jax import lax
from jax.experimental import pallas as pl
from jax.experimental.pallas import tpu as pltpu

N = 8192
D_IN = 768
E = 8
HID = 256
D_OUT = 256

BT2 = 2048
NT = N // BT2


def _ln(h):
    mu = jnp.mean(h, axis=-1, keepdims=True)
    var = jnp.mean((h - mu) ** 2, axis=-1, keepdims=True)
    return (h - mu) * lax.rsqrt(var + 1e-5)


def _gelu(h):
    return 0.5 * h * (1.0 + lax.erf(h * (2.0 ** -0.5)))


def _body(x_ref, wg1_ref, wg2_ref, W1_ref, b1_ref, g1_ref, be1_ref,
          W2_ref, b2_ref, g2_ref, be2_ref, W3_ref, b3_ref, o_ref, w_scr):
    e = pl.program_id(1)

    @pl.when(e == 0)
    def _():
        t = jnp.tanh(jnp.dot(x_ref[...], wg1_ref[...],
                             preferred_element_type=jnp.float32))
        logits = jnp.dot(t, wg2_ref[...], preferred_element_type=jnp.float32)
        m = jnp.max(logits, axis=-1, keepdims=True)
        ex = jnp.exp(logits - m)
        gw = ex / jnp.sum(ex, axis=-1, keepdims=True)
        a1 = jnp.argmax(gw, axis=-1)
        m1 = jnp.max(gw, axis=-1)
        lane = lax.broadcasted_iota(jnp.int32, gw.shape, 1)
        gw2 = jnp.where(lane == a1[:, None], -1.0, gw)
        a2 = jnp.argmax(gw2, axis=-1)
        m2 = jnp.max(gw2, axis=-1)
        s = m1 + m2 + 1e-12
        w_scr[...] = (jnp.where(lane == a1[:, None], (m1 / s)[:, None], 0.0)
                      + jnp.where(lane == a2[:, None], (m2 / s)[:, None], 0.0))

    xb = x_ref[...].astype(jnp.bfloat16)
    h = jnp.dot(xb, W1_ref[0].astype(jnp.bfloat16),
                preferred_element_type=jnp.float32) + b1_ref[0]
    h = _ln(h) * g1_ref[0] + be1_ref[0]
    h = _gelu(h)
    h = jnp.dot(h.astype(jnp.bfloat16), W2_ref[0].astype(jnp.bfloat16),
                preferred_element_type=jnp.float32) + b2_ref[0]
    h = _ln(h) * g2_ref[0] + be2_ref[0]
    h = _gelu(h)
    h = jnp.dot(h.astype(jnp.bfloat16), W3_ref[0].astype(jnp.bfloat16),
                preferred_element_type=jnp.float32) + b3_ref[0]
    wl = lax.broadcasted_iota(jnp.int32, (BT2, E), 1)
    wcol = jnp.sum(jnp.where(wl == e, w_scr[...], 0.0), axis=1, keepdims=True)
    contrib = h * wcol

    @pl.when(e == 0)
    def _():
        o_ref[...] = contrib

    @pl.when(e > 0)
    def _():
        o_ref[...] = o_ref[...] + contrib


def kernel(x, Wg1, Wg2, W1, b1, g1, be1, W2, b2, g2, be2, W3, b3):
    r3 = lambda a: a.reshape(E, 1, a.shape[-1])
    we = lambda s: pl.BlockSpec((1,) + s, lambda i, e, _n=len(s): (e,) + (0,) * _n)
    return pl.pallas_call(
        _body,
        grid=(NT, E),
        in_specs=[
            pl.BlockSpec((BT2, D_IN), lambda i, e: (i, 0)),
            pl.BlockSpec((D_IN, 2 * E), lambda i, e: (0, 0)),
            pl.BlockSpec((2 * E, E), lambda i, e: (0, 0)),
            we((D_IN, HID)), we((1, HID)), we((1, HID)), we((1, HID)),
            we((HID, HID)), we((1, HID)), we((1, HID)), we((1, HID)),
            we((HID, D_OUT)), we((1, D_OUT)),
        ],
        out_specs=pl.BlockSpec((BT2, D_OUT), lambda i, e: (i, 0)),
        out_shape=jax.ShapeDtypeStruct((N, D_OUT), jnp.float32),
        scratch_shapes=[pltpu.VMEM((BT2, E), jnp.float32)],
    )(x, Wg1, Wg2, W1, r3(b1), r3(g1), r3(be1), W2, r3(b2), r3(g2), r3(be2),
      W3, r3(b3))


# speedup vs baseline: 2.9687x; 1.0530x over previous
import jax
import jax.numpy as jnp
from jax import lax
from jax.experimental import pallas as pl
from jax.experimental.pallas import tpu as pltpu

N = 8192
D_IN = 768
E = 8
HID = 256
D_OUT = 256

BT2 = 4096
NT = N // BT2


def _ln(h):
    mu = jnp.mean(h, axis=-1, keepdims=True)
    var = jnp.mean((h - mu) ** 2, axis=-1, keepdims=True)
    return (h - mu) * lax.rsqrt(var + 1e-5)


def _gelu(h):
    return 0.5 * h * (1.0 + lax.erf(h * (2.0 ** -0.5)))


def _body(x_ref, wg1_ref, wg2_ref, W1_ref, b1_ref, g1_ref, be1_ref,
          W2_ref, b2_ref, g2_ref, be2_ref, W3_ref, b3_ref, o_ref, w_scr):
    e = pl.program_id(1)

    @pl.when(e == 0)
    def _():
        t = jnp.tanh(jnp.dot(x_ref[...], wg1_ref[...],
                             preferred_element_type=jnp.float32))
        logits = jnp.dot(t, wg2_ref[...], preferred_element_type=jnp.float32)
        m = jnp.max(logits, axis=-1, keepdims=True)
        ex = jnp.exp(logits - m)
        gw = ex / jnp.sum(ex, axis=-1, keepdims=True)
        a1 = jnp.argmax(gw, axis=-1)
        m1 = jnp.max(gw, axis=-1)
        lane = lax.broadcasted_iota(jnp.int32, gw.shape, 1)
        gw2 = jnp.where(lane == a1[:, None], -1.0, gw)
        a2 = jnp.argmax(gw2, axis=-1)
        m2 = jnp.max(gw2, axis=-1)
        s = m1 + m2 + 1e-12
        w_scr[...] = (jnp.where(lane == a1[:, None], (m1 / s)[:, None], 0.0)
                      + jnp.where(lane == a2[:, None], (m2 / s)[:, None], 0.0))

    xb = x_ref[...].astype(jnp.bfloat16)
    h = jnp.dot(xb, W1_ref[0].astype(jnp.bfloat16),
                preferred_element_type=jnp.float32) + b1_ref[0]
    h = _ln(h) * g1_ref[0] + be1_ref[0]
    h = _gelu(h)
    h = jnp.dot(h.astype(jnp.bfloat16), W2_ref[0].astype(jnp.bfloat16),
                preferred_element_type=jnp.float32) + b2_ref[0]
    h = _ln(h) * g2_ref[0] + be2_ref[0]
    h = _gelu(h)
    h = jnp.dot(h.astype(jnp.bfloat16), W3_ref[0].astype(jnp.bfloat16),
                preferred_element_type=jnp.float32) + b3_ref[0]
    wl = lax.broadcasted_iota(jnp.int32, (BT2, E), 1)
    wcol = jnp.sum(jnp.where(wl == e, w_scr[...], 0.0), axis=1, keepdims=True)
    contrib = h * wcol

    @pl.when(e == 0)
    def _():
        o_ref[...] = contrib

    @pl.when(e > 0)
    def _():
        o_ref[...] = o_ref[...] + contrib


def kernel(x, Wg1, Wg2, W1, b1, g1, be1, W2, b2, g2, be2, W3, b3):
    r3 = lambda a: a.reshape(E, 1, a.shape[-1])
    we = lambda s: pl.BlockSpec((1,) + s, lambda i, e, _n=len(s): (e,) + (0,) * _n)
    return pl.pallas_call(
        _body,
        grid=(NT, E),
        in_specs=[
            pl.BlockSpec((BT2, D_IN), lambda i, e: (i, 0)),
            pl.BlockSpec((D_IN, 2 * E), lambda i, e: (0, 0)),
            pl.BlockSpec((2 * E, E), lambda i, e: (0, 0)),
            we((D_IN, HID)), we((1, HID)), we((1, HID)), we((1, HID)),
            we((HID, HID)), we((1, HID)), we((1, HID)), we((1, HID)),
            we((HID, D_OUT)), we((1, D_OUT)),
        ],
        out_specs=pl.BlockSpec((BT2, D_OUT), lambda i, e: (i, 0)),
        out_shape=jax.ShapeDtypeStruct((N, D_OUT), jnp.float32),
        scratch_shapes=[pltpu.VMEM((BT2, E), jnp.float32)],
    )(x, Wg1, Wg2, W1, r3(b1), r3(g1), r3(be1), W2, r3(b2), r3(g2), r3(be2),
      W3, r3(b3))
